# Initial kernel scaffold; baseline (speedup 1.0000x reference)
import jax
import jax.numpy as jnp
from jax.experimental import pallas as pl


def kernel(x, a, W1, b1, W2, b2, W3, b3, Wd1, bd1, Wd2, bd2, Wo, bo):
    raise NotImplementedError("write your pallas kernel here")


# speedup vs baseline: 1.3312x; 1.3312x over previous
import jax
import jax.numpy as jnp
from jax.experimental import pallas as pl
from jax.experimental.pallas import tpu as pltpu


def _gcn_kernel(x_ref, a_ref, w1_ref, w2_ref, w3_ref, b1_ref, b2_ref, b3_ref,
                wd1_ref, bd1_ref, wd2_ref, bd2_ref, wo_ref, bo_ref,
                out_ref, a_sc, y_sc, h_sc, *, rb, nb):
    s = pl.program_id(1)
    i = pl.program_id(2)

    @pl.when(jnp.logical_and(s == 0, i == 0))
    def _():
        y_sc[...] = jnp.dot(x_ref[0].astype(jnp.bfloat16),
                            w1_ref[...].astype(jnp.bfloat16),
                            preferred_element_type=jnp.float32).astype(jnp.bfloat16)

    @pl.when(jnp.logical_and(s > 0, i == 0))
    def _():
        w = jnp.where(s == 1, w2_ref[...], w3_ref[...])
        y_sc[...] = jnp.dot(h_sc[...].astype(jnp.bfloat16),
                            w.astype(jnp.bfloat16),
                            preferred_element_type=jnp.float32).astype(jnp.bfloat16)

    bias = jnp.where(s == 0, b1_ref[...],
                     jnp.where(s == 1, b2_ref[...], b3_ref[...]))

    @pl.when(s == 0)
    def _():
        a_bf = a_ref[0].astype(jnp.bfloat16)
        a_sc[pl.ds(i * rb, rb), :] = a_bf
        z = jnp.dot(a_bf, y_sc[...], preferred_element_type=jnp.float32)
        h_sc[pl.ds(i * rb, rb), :] = jnp.maximum(z + bias, 0.0)

    @pl.when(s > 0)
    def _():
        a_bf = a_sc[pl.ds(i * rb, rb), :]
        z = jnp.dot(a_bf, y_sc[...], preferred_element_type=jnp.float32)
        h_sc[pl.ds(i * rb, rb), :] = jnp.maximum(z + bias, 0.0)

    @pl.when(jnp.logical_and(s == 2, i == nb - 1))
    def _():
        b = pl.program_id(0)
        p = jnp.mean(h_sc[...], axis=0, keepdims=True)
        p8 = jnp.broadcast_to(p, (8, p.shape[1])).astype(jnp.bfloat16)
        z1 = jnp.maximum(
            jnp.dot(p8, wd1_ref[...].astype(jnp.bfloat16),
                    preferred_element_type=jnp.float32) + bd1_ref[...], 0.0)
        z2 = jnp.maximum(
            jnp.dot(z1.astype(jnp.bfloat16), wd2_ref[...].astype(jnp.bfloat16),
                    preferred_element_type=jnp.float32) + bd2_ref[...], 0.0)
        logits = jnp.dot(z2.astype(jnp.bfloat16), wo_ref[...].astype(jnp.bfloat16),
                         preferred_element_type=jnp.float32) + bo_ref[...]
        m = jnp.max(logits, axis=-1, keepdims=True)
        e = jnp.exp(logits - m)
        sm = e / jnp.sum(e, axis=-1, keepdims=True)
        out_ref[pl.ds(b, 1), :] = sm[0:1, :]


def kernel(x, a, W1, b1, W2, b2, W3, b3, Wd1, bd1, Wd2, bd2, Wo, bo):
    B, N, F = x.shape
    H = W1.shape[1]
    L = Wo.shape[1]
    RB = 512
    NB = N // RB

    grid = (B, 3, NB)

    def full(arr):
        nd = arr.ndim
        return pl.BlockSpec(arr.shape, lambda b, s, i: (0,) * nd)

    b1r, b2r, b3r = b1.reshape(1, -1), b2.reshape(1, -1), b3.reshape(1, -1)
    bd1r, bd2r, bor = bd1.reshape(1, -1), bd2.reshape(1, -1), bo.reshape(1, -1)

    in_specs = [
        pl.BlockSpec((1, N, F), lambda b, s, i: (b, 0, 0)),
        pl.BlockSpec((1, RB, N),
                     lambda b, s, i: (b, jnp.where(s == 0, i, NB - 1), 0)),
        full(W1), full(W2), full(W3),
        full(b1r), full(b2r), full(b3r),
        full(Wd1), full(bd1r), full(Wd2), full(bd2r), full(Wo), full(bor),
    ]

    out = pl.pallas_call(
        lambda *refs: _gcn_kernel(*refs, rb=RB, nb=NB),
        grid=grid,
        in_specs=in_specs,
        out_specs=pl.BlockSpec((B, L), lambda b, s, i: (0, 0)),
        out_shape=jax.ShapeDtypeStruct((B, L), jnp.float32),
        scratch_shapes=[
            pltpu.VMEM((N, N), jnp.bfloat16),
            pltpu.VMEM((N, H), jnp.bfloat16),
            pltpu.VMEM((N, H), jnp.float32),
        ],
        compiler_params=pltpu.CompilerParams(
            dimension_semantics=("arbitrary", "arbitrary", "arbitrary"),
        ),
    )(x, a, W1, W2, W3, b1r, b2r, b3r, Wd1, bd1r, Wd2, bd2r, Wo, bor)
    return out
